# baseline (speedup 1.0000x reference)
import functools

import jax
import jax.numpy as jnp
from jax import lax
from jax.experimental import pallas as pl
from jax.experimental.pallas import tpu as pltpu
from jax.experimental.pallas import tpu_sc as plsc

BATCH = 4096
SEQ = 200
EMB = 64
HIDDEN = 256
CLASSES = 16

NUM_CORES = 2
NUM_SUBCORES = 16
LANES = 16
NW = NUM_CORES * NUM_SUBCORES
ROWS_PER_W = BATCH // NW
NBUF = 2
SPLIT = 128
REST = SEQ - SPLIT
VOCAB2 = 500000

_mesh = plsc.VectorSubcoreMesh(
    core_axis_name="c", subcore_axis_name="s",
    num_cores=NUM_CORES, num_subcores=NUM_SUBCORES)


@functools.partial(
    pl.kernel,
    mesh=_mesh,
    compiler_params=pltpu.CompilerParams(use_tc_tiling_on_sc=True),
    out_type=jax.ShapeDtypeStruct((BATCH, EMB), jnp.float32),
    scratch_types=[
        pltpu.VMEM((ROWS_PER_W * SEQ,), jnp.int32),
        pltpu.VMEM((ROWS_PER_W * SEQ,), jnp.int32),
        pltpu.VMEM((NBUF, SEQ, 2 * EMB), jnp.float32),
        pltpu.VMEM((ROWS_PER_W, EMB), jnp.float32),
        [pltpu.SemaphoreType.DMA] * NBUF,
    ],
)
def _pool(xh_hbm, xp_hbm, table_hbm, out_hbm, idx_v, par_v, rows_v, pool_v,
          sems):
    wid = lax.axis_index("s") * NUM_CORES + lax.axis_index("c")
    base = wid * ROWS_PER_W

    pltpu.sync_copy(xh_hbm.at[pl.ds(base * SEQ, ROWS_PER_W * SEQ)], idx_v)
    pltpu.sync_copy(xp_hbm.at[pl.ds(base * SEQ, ROWS_PER_W * SEQ)], par_v)

    def issue(r, slot):
        pltpu.make_async_copy(
            table_hbm.at[idx_v.at[pl.ds(r * SEQ, SPLIT)]],
            rows_v.at[slot, pl.ds(0, SPLIT)],
            sems[slot]).start()
        pltpu.make_async_copy(
            table_hbm.at[idx_v.at[pl.ds(r * SEQ + SPLIT, REST)]],
            rows_v.at[slot, pl.ds(SPLIT, REST)],
            sems[slot]).start()

    def wait_slot(slot):
        pltpu.make_async_copy(
            table_hbm.at[pl.ds(0, SEQ)], rows_v.at[slot], sems[slot]).wait()

    def reduce_row(slot, r):
        nvec = EMB // LANES

        def step(i0, k, accs):
            p = jnp.full((LANES,), accs[nvec][k], jnp.float32)
            for c in range(nvec):
                lo = rows_v[slot, i0 + k, pl.ds(LANES * c, LANES)]
                hi = rows_v[slot, i0 + k, pl.ds(EMB + LANES * c, LANES)]
                accs[c] = accs[c] + lo + p * (hi - lo)
            return accs

        def body2(g, accs):
            i0 = g * LANES
            par16 = par_v[pl.ds(r * SEQ + i0, LANES)].astype(jnp.float32)
            accs = list(accs) + [par16]
            for k in range(LANES):
                accs = step(i0, k, accs)
            return tuple(accs[:nvec])

        zero = jnp.zeros((LANES,), jnp.float32)
        accs = lax.fori_loop(0, SEQ // LANES, body2, (zero,) * nvec)
        tail0 = SEQ - LANES
        par16 = par_v[pl.ds(r * SEQ + tail0, LANES)].astype(jnp.float32)
        accs = list(accs) + [par16]
        for k in range(LANES - (SEQ - (SEQ // LANES) * LANES), LANES):
            accs = step(tail0, k, accs)
        accs = accs[:nvec]
        for c in range(nvec):
            pool_v[r, pl.ds(LANES * c, LANES)] = accs[c] * (1.0 / SEQ)

    for p in range(NBUF - 1):
        issue(p, p)

    def outer(g, _):
        for b in range(NBUF):
            r = g * NBUF + b
            nxt = r + NBUF - 1

            @pl.when(nxt < ROWS_PER_W)
            def _():
                issue(nxt, (b + NBUF - 1) % NBUF)

            wait_slot(b)
            reduce_row(b, r)
        return 0

    lax.fori_loop(0, ROWS_PER_W // NBUF, outer, 0)
    pltpu.sync_copy(pool_v, out_hbm.at[pl.ds(base, ROWS_PER_W)])


def _mlp_body(x_ref, w1_ref, b1_ref, w2_ref, b2_ref, o_ref):
    h = jnp.dot(x_ref[...], w1_ref[...],
                preferred_element_type=jnp.float32) + b1_ref[...]
    logits = jnp.dot(h, w2_ref[...],
                     preferred_element_type=jnp.float32) + b2_ref[...]
    m = jnp.max(logits, axis=-1, keepdims=True)
    e = jnp.exp(logits - m)
    o_ref[...] = e / jnp.sum(e, axis=-1, keepdims=True)


_mlp = pl.pallas_call(
    _mlp_body,
    out_shape=jax.ShapeDtypeStruct((BATCH, CLASSES), jnp.float32),
)


def kernel(x, emb_table, W1, b1, W2, b2):
    xi = x.astype(jnp.int32).reshape(-1)
    xh = xi // 2
    xp = xi % 2
    tbl2 = emb_table.reshape(VOCAB2, 2 * EMB)
    pooled = _pool(xh, xp, tbl2)
    return _mlp(pooled, W1, b1.reshape(1, HIDDEN), W2, b2.reshape(1, CLASSES))
